# Initial kernel scaffold; baseline (speedup 1.0000x reference)
import jax
import jax.numpy as jnp
from jax.experimental import pallas as pl


def kernel(x, w0, b0, g0, be0, w1, b1, g1, be1, w2, b2, g2, be2, wr, br, gr, ber):
    raise NotImplementedError("write your pallas kernel here")


# speedup vs baseline: 11.3925x; 11.3925x over previous
import functools

import jax
import jax.numpy as jnp
from jax import lax
from jax.experimental import pallas as pl
from jax.experimental.pallas import tpu as pltpu
from jax.experimental.pallas import tpu_sc as plsc

B, N, S = 16, 4096, 512
K_SMALL, K_BIG = 16, 32
BN_EPS = 1e-5
S_TILE = 256
GRID = (B * S) // S_TILE


def _fps_body(x0_ref, x1_ref, x2_ref, c0_ref, c1_ref, c2_ref, dmin_ref):
    x0 = x0_ref[...]
    x1 = x1_ref[...]
    x2 = x2_ref[...]
    iota_n = lax.broadcasted_iota(jnp.int32, (B, N), 1)
    iota_s = lax.broadcasted_iota(jnp.int32, (B, S), 1)
    dmin_ref[...] = jnp.full((B, N), 1e10, jnp.float32)

    def body(i, far):
        mask = iota_n == far
        c0 = jnp.sum(jnp.where(mask, x0, 0.0), axis=1, keepdims=True)
        c1 = jnp.sum(jnp.where(mask, x1, 0.0), axis=1, keepdims=True)
        c2 = jnp.sum(jnp.where(mask, x2, 0.0), axis=1, keepdims=True)
        sel = iota_s == i
        c0_ref[...] = jnp.where(sel, c0, c0_ref[...])
        c1_ref[...] = jnp.where(sel, c1, c1_ref[...])
        c2_ref[...] = jnp.where(sel, c2, c2_ref[...])
        d = (x0 - c0) ** 2 + (x1 - c1) ** 2 + (x2 - c2) ** 2
        dm = jnp.minimum(dmin_ref[...], d)
        dmin_ref[...] = dm
        m = jnp.max(dm, axis=1, keepdims=True)
        far_new = jnp.min(jnp.where(dm == m, iota_n, N), axis=1, keepdims=True)
        return far_new.astype(jnp.int32)

    lax.fori_loop(0, S, body, jnp.zeros((B, 1), jnp.int32))


def _fps(x0, x1, x2):
    out = [jax.ShapeDtypeStruct((B, S), jnp.float32)] * 3
    return pl.pallas_call(
        _fps_body,
        out_shape=out,
        scratch_shapes=[pltpu.VMEM((B, N), jnp.float32)],
    )(x0, x1, x2)


def _topk_body(c_ref, x_ref, idx_ref, d_ref):
    c = c_ref[...].reshape(S, 3)
    xm = x_ref[...].reshape(N, 3)
    dot = lax.dot_general(c, xm, (((1,), (1,)), ((), ())),
                          preferred_element_type=jnp.float32)
    cn = jnp.sum(c * c, axis=1, keepdims=True)
    xn = jnp.sum(xm * xm, axis=1, keepdims=True).reshape(1, N)
    d_ref[...] = cn + xn - 2.0 * dot
    iota_n = lax.broadcasted_iota(jnp.int32, (S, N), 1)
    iota_k = lax.broadcasted_iota(jnp.int32, (S, K_BIG), 1)

    def body(k, acc):
        d = d_ref[...]
        m = jnp.min(d, axis=1, keepdims=True)
        idx = jnp.min(jnp.where(d == m, iota_n, N), axis=1, keepdims=True)
        acc = jnp.where(iota_k == k, idx, acc)
        d_ref[...] = jnp.where(iota_n == idx, jnp.inf, d)
        return acc

    acc = lax.fori_loop(0, K_BIG, body,
                        jnp.zeros((S, K_BIG), jnp.int32))
    idx_ref[...] = acc.reshape(1, S, K_BIG)


def _topk(cmat, x):
    return pl.pallas_call(
        _topk_body,
        grid=(B,),
        in_specs=[pl.BlockSpec((1, S, 3), lambda b: (b, 0, 0)),
                  pl.BlockSpec((1, N, 3), lambda b: (b, 0, 0))],
        out_specs=pl.BlockSpec((1, S, K_BIG), lambda b: (b, 0, 0)),
        out_shape=jax.ShapeDtypeStruct((B, S, K_BIG), jnp.int32),
        scratch_shapes=[pltpu.VMEM((S, N), jnp.float32)],
    )(cmat, x)


_SC_BCHUNK = 8
_SC_TPB = 32 // _SC_BCHUNK
_SC_SLOT = S // _SC_TPB


def _sc_gather_body(x0_hbm, x1_hbm, x2_hbm, idx_hbm,
                    g0_hbm, g1_hbm, g2_hbm,
                    xv0, xv1, xv2, idxv, ov0, ov1, ov2):
    nc = 2
    wid = lax.axis_index("s") * nc + lax.axis_index("c")
    b = wid // _SC_TPB
    part = wid % _SC_TPB
    s0 = part * _SC_SLOT
    pltpu.sync_copy(x0_hbm.at[b], xv0)
    pltpu.sync_copy(x1_hbm.at[b], xv1)
    pltpu.sync_copy(x2_hbm.at[b], xv2)
    pltpu.sync_copy(idx_hbm.at[b, pl.ds(s0, _SC_SLOT)], idxv)

    def body(j, carry):
        for h in range(K_BIG // 16):
            iv = idxv[j, pl.ds(h * 16, 16)]
            ov0[j, pl.ds(h * 16, 16)] = plsc.load_gather(xv0, [iv])
            ov1[j, pl.ds(h * 16, 16)] = plsc.load_gather(xv1, [iv])
            ov2[j, pl.ds(h * 16, 16)] = plsc.load_gather(xv2, [iv])
        return carry

    lax.fori_loop(0, _SC_SLOT, body, 0)
    pltpu.sync_copy(ov0, g0_hbm.at[b, pl.ds(s0, _SC_SLOT)])
    pltpu.sync_copy(ov1, g1_hbm.at[b, pl.ds(s0, _SC_SLOT)])
    pltpu.sync_copy(ov2, g2_hbm.at[b, pl.ds(s0, _SC_SLOT)])


def _sc_gather(x0, x1, x2, idx):
    mesh = plsc.VectorSubcoreMesh(core_axis_name="c", subcore_axis_name="s")
    out = [jax.ShapeDtypeStruct((_SC_BCHUNK, S, K_BIG), jnp.float32)] * 3
    f = pl.kernel(
        _sc_gather_body,
        mesh=mesh,
        out_type=out,
        compiler_params=pltpu.CompilerParams(needs_layout_passes=False),
        scratch_types=[
            pltpu.VMEM((N,), jnp.float32),
            pltpu.VMEM((N,), jnp.float32),
            pltpu.VMEM((N,), jnp.float32),
            pltpu.VMEM((_SC_SLOT, K_BIG), jnp.int32),
            pltpu.VMEM((_SC_SLOT, K_BIG), jnp.float32),
            pltpu.VMEM((_SC_SLOT, K_BIG), jnp.float32),
            pltpu.VMEM((_SC_SLOT, K_BIG), jnp.float32),
        ],
    )
    parts = [f(x0[c:c + _SC_BCHUNK], x1[c:c + _SC_BCHUNK],
               x2[c:c + _SC_BCHUNK], idx[c:c + _SC_BCHUNK])
             for c in range(0, B, _SC_BCHUNK)]
    return tuple(jnp.concatenate([p[i] for p in parts], axis=0)
                 for i in range(3))


def _acc_out(ref, val):
    @pl.when(pl.program_id(0) == 0)
    def _():
        ref[...] = val

    @pl.when(pl.program_id(0) != 0)
    def _():
        ref[...] = ref[...] + val


def _nt_dot(g, w):
    return lax.dot_general(g, w, (((1,), (1,)), ((), ())),
                           preferred_element_type=jnp.float32)


def _g0_flat(k, gm_ref, c_ref):
    g = gm_ref[...] - c_ref[...][:, None, :]
    return g.reshape(S_TILE * k, 3)


def _layer(g, w_ref, b_ref, sc_ref, off_ref):
    z = _nt_dot(g, w_ref[...]) + b_ref[...]
    return jnp.maximum(z * sc_ref[...] + off_ref[...], 0.0)


def _zstat(z, s_ref, q_ref):
    _acc_out(s_ref, jnp.sum(z, axis=0, keepdims=True))
    _acc_out(q_ref, jnp.sum(z * z, axis=0, keepdims=True))


def _zstat1_body(k, gm_ref, c_ref, w0, b0r, s_ref, q_ref):
    z1 = _nt_dot(_g0_flat(k, gm_ref, c_ref), w0[...]) + b0r[...]
    _zstat(z1, s_ref, q_ref)


def _zstat2_body(k, gm_ref, c_ref, w0, b0r, sc1, of1, w1, b1r,
                 s_ref, q_ref):
    g1 = _layer(_g0_flat(k, gm_ref, c_ref), w0, b0r, sc1, of1)
    z2 = _nt_dot(g1, w1[...]) + b1r[...]
    _zstat(z2, s_ref, q_ref)


def _zstat3_body(k, gm_ref, c_ref, w0, b0r, sc1, of1, w1, b1r, sc2, of2,
                 w2, b2r, s_ref, q_ref):
    g1 = _layer(_g0_flat(k, gm_ref, c_ref), w0, b0r, sc1, of1)
    g2 = _layer(g1, w1, b1r, sc2, of2)
    z3 = _nt_dot(g2, w2[...]) + b2r[...]
    _zstat(z3, s_ref, q_ref)


def _layer3_body(k, gm_ref, c_ref, w0, b0r, sc1, of1, w1, b1r, sc2, of2,
                 w2, b2r, sc3, of3, wr, brr,
                 p_ref, h_ref, hs_ref, hq_ref):
    g1 = _layer(_g0_flat(k, gm_ref, c_ref), w0, b0r, sc1, of1)
    g2 = _layer(g1, w1, b1r, sc2, of2)
    g3 = _layer(g2, w2, b2r, sc3, of3)
    patches = jnp.max(g3.reshape(S_TILE, k, 128), axis=1)
    h = _nt_dot(patches, wr[...]) + brr[...]
    p_ref[...] = patches
    h_ref[...] = h
    _acc_out(hs_ref, jnp.sum(h, axis=0, keepdims=True))
    _acc_out(hq_ref, jnp.sum(h * h, axis=0, keepdims=True))


def _final_body(p_ref, h_ref, sc_ref, off_ref, out_ref):
    out_ref[...] = jnp.maximum(
        h_ref[...] * sc_ref[...] + off_ref[...] + p_ref[...], 0.0)


def _tile_specs(k):
    gspec = pl.BlockSpec((S_TILE, k, 3), lambda i: (i, 0, 0))
    cspec = pl.BlockSpec((S_TILE, 3), lambda i: (i, 0))
    return [gspec, cspec]


def _full(shape):
    return pl.BlockSpec(shape, lambda i: tuple(0 for _ in shape))


def _affine_specs(cout, cin):
    return [_full((cout, cin)), _full((1, cout)),
            _full((1, cout)), _full((1, cout))]


def _stat_out(c):
    return dict(
        out_specs=[_full((1, c)), _full((1, c))],
        out_shape=[jax.ShapeDtypeStruct((1, c), jnp.float32),
                   jax.ShapeDtypeStruct((1, c), jnp.float32)],
    )


def _zstat1(k, gm, crows, w0, b0r):
    f = pl.pallas_call(
        functools.partial(_zstat1_body, k),
        grid=(GRID,),
        in_specs=_tile_specs(k) + [_full((32, 3)), _full((1, 32))],
        **_stat_out(32),
    )
    return f(gm, crows, w0, b0r)


def _zstat2(k, gm, crows, l1, w1, b1r):
    f = pl.pallas_call(
        functools.partial(_zstat2_body, k),
        grid=(GRID,),
        in_specs=_tile_specs(k) + _affine_specs(32, 3)
        + [_full((64, 32)), _full((1, 64))],
        **_stat_out(64),
    )
    return f(gm, crows, *l1, w1, b1r)


def _zstat3(k, gm, crows, l1, l2, w2, b2r):
    f = pl.pallas_call(
        functools.partial(_zstat3_body, k),
        grid=(GRID,),
        in_specs=_tile_specs(k) + _affine_specs(32, 3) + _affine_specs(64, 32)
        + [_full((128, 64)), _full((1, 128))],
        **_stat_out(128),
    )
    return f(gm, crows, *l1, *l2, w2, b2r)


def _layer3(k, gm, crows, l1, l2, l3, wr, brr):
    f = pl.pallas_call(
        functools.partial(_layer3_body, k),
        grid=(GRID,),
        in_specs=_tile_specs(k) + _affine_specs(32, 3)
        + _affine_specs(64, 32) + _affine_specs(128, 64)
        + [_full((128, 128)), _full((1, 128))],
        out_specs=[pl.BlockSpec((S_TILE, 128), lambda i: (i, 0)),
                   pl.BlockSpec((S_TILE, 128), lambda i: (i, 0)),
                   _full((1, 128)), _full((1, 128))],
        out_shape=[jax.ShapeDtypeStruct((B * S, 128), jnp.float32),
                   jax.ShapeDtypeStruct((B * S, 128), jnp.float32),
                   jax.ShapeDtypeStruct((1, 128), jnp.float32),
                   jax.ShapeDtypeStruct((1, 128), jnp.float32)],
    )
    return f(gm, crows, *l1, *l2, *l3, wr, brr)


def _final(p, h, scale, off):
    f = pl.pallas_call(
        _final_body,
        grid=(GRID,),
        in_specs=[pl.BlockSpec((S_TILE, 128), lambda i: (i, 0)),
                  pl.BlockSpec((S_TILE, 128), lambda i: (i, 0)),
                  _full((1, 128)), _full((1, 128))],
        out_specs=pl.BlockSpec((S_TILE, 128), lambda i: (i, 0)),
        out_shape=jax.ShapeDtypeStruct((B * S, 128), jnp.float32),
    )
    return f(p, h, scale, off)


def _fold(gamma, beta, s, q, n):
    mu = s / n
    var = q / n - mu * mu
    scale = gamma / jnp.sqrt(var + BN_EPS)
    off = beta - mu * scale
    return scale.reshape(1, -1), off.reshape(1, -1)


def _branch(k, gm, crows, w0, b0, gam0, be0, w1, b1, gam1, be1,
            w2, b2, gam2, be2, wr, br, gr, ber):
    n = float(B * S * k)
    b0r = b0.reshape(1, -1)
    b1r = b1.reshape(1, -1)
    b2r = b2.reshape(1, -1)
    s1, q1 = _zstat1(k, gm, crows, w0, b0r)
    sc1, of1 = _fold(gam0, be0, s1[0], q1[0], n)
    l1 = (w0, b0r, sc1, of1)
    s2, q2 = _zstat2(k, gm, crows, l1, w1, b1r)
    sc2, of2 = _fold(gam1, be1, s2[0], q2[0], n)
    l2 = (w1, b1r, sc2, of2)
    s3, q3 = _zstat3(k, gm, crows, l1, l2, w2, b2r)
    sc3, of3 = _fold(gam2, be2, s3[0], q3[0], n)
    l3 = (w2, b2r, sc3, of3)
    patches, h, hs, hq = _layer3(k, gm, crows, l1, l2, l3,
                                 wr, br.reshape(1, -1))
    ns = float(B * S)
    mu = hs[0] / ns
    var = hq[0] / ns - mu * mu
    scale = gr / jnp.sqrt(var + BN_EPS)
    off = ber - mu * scale
    out = _final(patches, h, scale.reshape(1, -1), off.reshape(1, -1))
    return out.reshape(B, S, 128)


def kernel(x, w0, b0, g0, be0, w1, b1, g1, be1, w2, b2, g2, be2,
           wr, br, gr, ber):
    xt = jnp.transpose(x, (2, 0, 1))
    x0, x1, x2 = xt[0], xt[1], xt[2]
    c0, c1, c2 = _fps(x0, x1, x2)
    cmat = jnp.stack([c0, c1, c2], axis=-1)
    idx = _topk(cmat, x)
    gx, gy, gz = _sc_gather(x0, x1, x2, idx)

    gmat = jnp.stack([gx, gy, gz], axis=-1).reshape(B * S, K_BIG, 3)
    crows = cmat.reshape(B * S, 3)

    outs = [cmat]
    for k in (K_SMALL, K_BIG):
        outs.append(_branch(k, gmat[:, :k, :], crows,
                            w0, b0, g0, be0, w1, b1, g1, be1,
                            w2, b2, g2, be2, wr, br, gr, ber))
    return jnp.concatenate(outs, axis=-1)
